# baseline (speedup 1.0000x reference)
import functools

import jax
import jax.numpy as jnp
from jax import lax
from jax.experimental import pallas as pl
from jax.experimental.pallas import tpu as pltpu
from jax.experimental.pallas import tpu_sc as plsc

B, S, D = 4, 2048, 64
W = 128
NW = S // W
NCOL = 16
HALF = S // 2


def _perm(x, idx):
    dn = lax.GatherDimensionNumbers(
        offset_dims=(), collapsed_slice_dims=(0,), start_index_map=(0,))
    return lax.gather(x, idx[:, None], dn, (1,),
                      mode=lax.GatherScatterMode.PROMISE_IN_BOUNDS)


def _bitonic_local(xs):
    n = len(xs)
    if n == 1:
        return [lax.sort(xs[0], dimension=0)]
    h = n // 2
    lo = [jnp.minimum(xs[i], xs[i + h]) for i in range(h)]
    hi = [jnp.maximum(xs[i], xs[i + h]) for i in range(h)]
    return _bitonic_local(lo) + _bitonic_local(hi)


def _bmerge(a, b):
    n = len(a)
    br = [lax.rev(x, (0,)) for x in reversed(b)]
    lo = [jnp.minimum(a[i], br[i]) for i in range(n)]
    hi = [jnp.maximum(a[i], br[i]) for i in range(n)]
    return _bitonic_local(lo) + _bitonic_local(hi)


def _sort128(vs):
    runs = [[lax.sort(v, dimension=0)] for v in vs]
    while len(runs) > 1:
        runs = [_bmerge(runs[i], runs[i + 1])
                for i in range(0, len(runs), 2)]
    return runs[0]


def _col_geometry(d):
    dm1 = d - 1
    is0 = d == 0
    f = jnp.where(is0, 0, lax.rem(dm1, 11))
    o = jnp.bitwise_and(dm1, 15)
    st_full = jnp.where(is0, 0, 1 << f)
    off = jnp.where(is0, 0, o * 128)
    st_win = jnp.where(is0, 0, jnp.where(f <= 6, 1 << f, 0))
    whi = jnp.where(f >= 7, 1 << jnp.maximum(f - 7, 0), 0)
    whi = jnp.where(is0, 0, whi)
    return off, st_full, st_win, whi, o, is0


_mesh = plsc.VectorSubcoreMesh(core_axis_name="c", subcore_axis_name="s")


@functools.partial(
    pl.kernel,
    out_type=jax.ShapeDtypeStruct((B, D, S), jnp.float32),
    mesh=_mesh,
    compiler_params=pltpu.CompilerParams(
        needs_layout_passes=False, skip_device_barrier=True),
    scratch_types=[
        pltpu.VMEM((NCOL, S), jnp.float32),
        pltpu.VMEM((HALF,), jnp.float32),
        pltpu.VMEM((HALF,), jnp.float32),
        pltpu.VMEM((HALF,), jnp.float32),
        pltpu.VMEM((W,), jnp.float32),
        pltpu.VMEM((HALF,), jnp.float32),
        pltpu.VMEM((NW // 2, NCOL, W), jnp.float32),
        pltpu.SemaphoreType.DMA((NW // 2,)),
        pltpu.VMEM_SHARED((B, 4, HALF), jnp.float32),
        pltpu.VMEM_SHARED((B, HALF), jnp.float32),
    ],
)
def _swd_sc(v_hbm, out_hbm, cols_v, m_buf, tmp_buf, a_buf, sa_buf, c0_buf,
            out_st, osem, m_sh, c0_sh):
    c = lax.axis_index("c")
    sid = lax.axis_index("s")
    b = sid // 4
    cg = sid % 4
    wh = c
    iota = lax.iota(jnp.int32, 16)

    pltpu.sync_copy(v_hbm.at[b, pl.ds(cg * NCOL, NCOL), :], cols_v)

    geo = [_col_geometry(cg * NCOL + j) for j in range(NCOL)]
    jcols = [jnp.full((16,), 0, jnp.int32) + j for j in range(NCOL)]

    @plsc.parallel_loop(0, HALF // 16, unroll=2)
    def _(ch):
        svec = wh * HALF + ch * 16 + iota
        acc = jnp.zeros((16,), jnp.float32)
        for j in range(NCOL):
            off, st_full, _, _, _, _ = geo[j]
            rows = jnp.bitwise_and(svec - off, S - 1) ^ st_full
            g = plsc.load_gather(cols_v, [jcols[j], rows])
            acc = jnp.maximum(acc, jnp.abs(g))
        m_buf[pl.ds(ch * 16, 16)] = acc
    pltpu.sync_copy(m_buf, m_sh.at[b, cg])

    @pl.when(cg == 0)
    def _():
        pltpu.sync_copy(cols_v.at[0, pl.ds(wh * HALF, HALF)], c0_sh.at[b])

    plsc.subcore_barrier()

    pltpu.sync_copy(m_sh.at[b, 0], m_buf)
    for cgi in range(1, 4):
        pltpu.sync_copy(m_sh.at[b, cgi], tmp_buf)

        def mx_chunk(ch, _):
            o = ch * 16
            m_buf[pl.ds(o, 16)] = jnp.maximum(m_buf[pl.ds(o, 16)],
                                              tmp_buf[pl.ds(o, 16)])
            return 0

        lax.fori_loop(0, HALF // 16, mx_chunk, 0)
    pltpu.sync_copy(c0_sh.at[b], c0_buf)

    def a_chunk(ch, _):
        o = ch * 16
        c0v = c0_buf[pl.ds(o, 16)]
        sgn = jnp.sign(c0v - _perm(c0v, iota ^ 1))
        a_buf[pl.ds(o, 16)] = sgn * m_buf[pl.ds(o, 16)]
        return 0

    lax.fori_loop(0, HALF // 16, a_chunk, 0)

    def out_slab(wg):
        return out_hbm.at[b, pl.ds(cg * NCOL, NCOL), pl.ds(wg * W, W)]

    @plsc.parallel_loop(0, NW // 2, unroll=2)
    def _(w):
        wg = wh * (NW // 2) + w

        avs = [a_buf[pl.ds(w * W + i * 16, 16)] for i in range(8)]
        sav = tuple(_sort128(avs))

        @plsc.parallel_loop(0, NCOL, unroll=2, carry=sav)
        def _(j, sa):
            d = cg * NCOL + j
            _, _, st_win, whi, o, is0 = _col_geometry(d)
            wsrc = jnp.where(is0, wg, jnp.bitwise_and(wg - o, NW - 1) ^ whi)
            st_lo = jnp.bitwise_and(st_win, 15)
            st_hi = st_win >> 4
            alt = []
            for i in range(8):
                src_blk = i ^ st_hi
                av = a_buf[pl.ds(w * W + src_blk * 16, 16)]
                ap = _perm(av, iota ^ st_lo)
                rv = cols_v[j, pl.ds(wsrc * W + i * 16, 16)]
                alt.append(rv + ap)
            srt = _sort128(alt)
            for i in range(8):
                out_st[w, j, pl.ds(i * 16, 16)] = srt[i] - sa[i]
            return sa

        pltpu.async_copy(out_st.at[w], out_slab(wg), osem.at[w])

    for t in range(NW // 2):
        wg = wh * (NW // 2) + t
        pltpu.make_async_copy(out_st.at[t], out_slab(wg),
                              osem.at[t]).wait()


def kernel(q, k, v):
    del q, k
    v_t = jnp.transpose(v, (0, 2, 1))
    out_t = _swd_sc(v_t)
    return jnp.transpose(out_t, (0, 2, 1))


# speedup vs baseline: 1.0310x; 1.0310x over previous
import functools

import jax
import jax.numpy as jnp
from jax import lax
from jax.experimental import pallas as pl
from jax.experimental.pallas import tpu as pltpu
from jax.experimental.pallas import tpu_sc as plsc

B, S, D = 4, 2048, 64
W = 128
NW = S // W
NCOL = 16
HALF = S // 2


def _perm(x, idx):
    dn = lax.GatherDimensionNumbers(
        offset_dims=(), collapsed_slice_dims=(0,), start_index_map=(0,))
    return lax.gather(x, idx[:, None], dn, (1,),
                      mode=lax.GatherScatterMode.PROMISE_IN_BOUNDS)


def _bitonic_local(xs):
    n = len(xs)
    if n == 1:
        return [lax.sort(xs[0], dimension=0)]
    h = n // 2
    lo = [jnp.minimum(xs[i], xs[i + h]) for i in range(h)]
    hi = [jnp.maximum(xs[i], xs[i + h]) for i in range(h)]
    return _bitonic_local(lo) + _bitonic_local(hi)


def _bmerge(a, b):
    n = len(a)
    br = [lax.rev(x, (0,)) for x in reversed(b)]
    lo = [jnp.minimum(a[i], br[i]) for i in range(n)]
    hi = [jnp.maximum(a[i], br[i]) for i in range(n)]
    return _bitonic_local(lo) + _bitonic_local(hi)


def _sort128(vs):
    runs = [[lax.sort(v, dimension=0)] for v in vs]
    while len(runs) > 1:
        runs = [_bmerge(runs[i], runs[i + 1])
                for i in range(0, len(runs), 2)]
    return runs[0]


def _col_geometry(d):
    dm1 = d - 1
    is0 = d == 0
    f = jnp.where(is0, 0, lax.rem(dm1, 11))
    o = jnp.bitwise_and(dm1, 15)
    st_full = jnp.where(is0, 0, 1 << f)
    off = jnp.where(is0, 0, o * 128)
    st_win = jnp.where(is0, 0, jnp.where(f <= 6, 1 << f, 0))
    whi = jnp.where(f >= 7, 1 << jnp.maximum(f - 7, 0), 0)
    whi = jnp.where(is0, 0, whi)
    return off, st_full, st_win, whi, o, is0


_mesh = plsc.VectorSubcoreMesh(core_axis_name="c", subcore_axis_name="s")


@functools.partial(
    pl.kernel,
    out_type=jax.ShapeDtypeStruct((B, D, S), jnp.float32),
    mesh=_mesh,
    compiler_params=pltpu.CompilerParams(needs_layout_passes=False),
    scratch_types=[
        pltpu.VMEM((NCOL, S), jnp.float32),
        pltpu.VMEM((HALF,), jnp.float32),
        pltpu.VMEM((HALF,), jnp.float32),
        pltpu.VMEM((HALF,), jnp.float32),
        pltpu.VMEM((HALF,), jnp.float32),
        pltpu.VMEM((NW // 2, NCOL, W), jnp.float32),
        pltpu.SemaphoreType.DMA((NW // 2,)),
        pltpu.VMEM_SHARED((B, 4, HALF), jnp.float32),
        pltpu.VMEM_SHARED((B, HALF), jnp.float32),
    ],
)
def _swd_sc(v_hbm, out_hbm, cols_v, m_buf, tmp_buf, a_buf, c0_buf,
            out_st, osem, m_sh, c0_sh):
    c = lax.axis_index("c")
    sid = lax.axis_index("s")
    b = sid // 4
    cg = sid % 4
    wh = c
    iota = lax.iota(jnp.int32, 16)

    pltpu.sync_copy(v_hbm.at[b, pl.ds(cg * NCOL, NCOL), :], cols_v)

    geo = [_col_geometry(cg * NCOL + j) for j in range(NCOL)]
    jcols = [jnp.full((16,), 0, jnp.int32) + j for j in range(NCOL)]

    @plsc.parallel_loop(0, HALF // 16, unroll=2)
    def _(ch):
        svec = wh * HALF + ch * 16 + iota
        acc = jnp.zeros((16,), jnp.float32)
        for j in range(NCOL):
            off, st_full, _, _, _, _ = geo[j]
            rows = jnp.bitwise_and(svec - off, S - 1) ^ st_full
            g = plsc.load_gather(cols_v, [jcols[j], rows])
            acc = jnp.maximum(acc, jnp.abs(g))
        m_buf[pl.ds(ch * 16, 16)] = acc
    pltpu.sync_copy(m_buf, m_sh.at[b, cg])

    @pl.when(cg == 0)
    def _():
        pltpu.sync_copy(cols_v.at[0, pl.ds(wh * HALF, HALF)], c0_sh.at[b])

    plsc.subcore_barrier()

    pltpu.sync_copy(m_sh.at[b, 0], m_buf)
    for cgi in range(1, 4):
        pltpu.sync_copy(m_sh.at[b, cgi], tmp_buf)

        def mx_chunk(ch, _):
            o = ch * 16
            m_buf[pl.ds(o, 16)] = jnp.maximum(m_buf[pl.ds(o, 16)],
                                              tmp_buf[pl.ds(o, 16)])
            return 0

        lax.fori_loop(0, HALF // 16, mx_chunk, 0)
    pltpu.sync_copy(c0_sh.at[b], c0_buf)

    def a_chunk(ch, _):
        o = ch * 16
        c0v = c0_buf[pl.ds(o, 16)]
        sgn = jnp.sign(c0v - _perm(c0v, iota ^ 1))
        a_buf[pl.ds(o, 16)] = sgn * m_buf[pl.ds(o, 16)]
        return 0

    lax.fori_loop(0, HALF // 16, a_chunk, 0)

    def out_slab(wg):
        return out_hbm.at[b, pl.ds(cg * NCOL, NCOL), pl.ds(wg * W, W)]

    @plsc.parallel_loop(0, NW // 2)
    def _(w):
        wg = wh * (NW // 2) + w

        avs = [a_buf[pl.ds(w * W + i * 16, 16)] for i in range(8)]
        sav = tuple(_sort128(avs))

        @plsc.parallel_loop(0, NCOL, carry=sav)
        def _(j, sa):
            d = cg * NCOL + j
            _, _, st_win, whi, o, is0 = _col_geometry(d)
            wsrc = jnp.where(is0, wg, jnp.bitwise_and(wg - o, NW - 1) ^ whi)
            st_lo = jnp.bitwise_and(st_win, 15)
            st_hi = st_win >> 4
            alt = []
            for i in range(8):
                src_blk = i ^ st_hi
                av = a_buf[pl.ds(w * W + src_blk * 16, 16)]
                ap = _perm(av, iota ^ st_lo)
                rv = cols_v[j, pl.ds(wsrc * W + i * 16, 16)]
                alt.append(rv + ap)
            srt = _sort128(alt)
            for i in range(8):
                out_st[w, j, pl.ds(i * 16, 16)] = srt[i] - sa[i]
            return sa

        pltpu.async_copy(out_st.at[w], out_slab(wg), osem.at[w])

    for t in range(NW // 2):
        wg = wh * (NW // 2) + t
        pltpu.make_async_copy(out_st.at[t], out_slab(wg),
                              osem.at[t]).wait()


def kernel(q, k, v):
    del q, k
    v_t = jnp.transpose(v, (0, 2, 1))
    out_t = _swd_sc(v_t)
    return jnp.transpose(out_t, (0, 2, 1))
